# Initial kernel scaffold; baseline (speedup 1.0000x reference)
import functools

import jax
import jax.numpy as jnp
from jax import lax
from jax.experimental import pallas as pl
from jax.experimental.pallas import tpu as pltpu
from jax.experimental.pallas import tpu_sc as plsc

N = 100000
E = 3200000
D_IN = 128
H1 = 64
H2 = 16

NC = 2
NS = 16
NW = NC * NS

C = 2048
K = C // 128
EPAD = 49 * NW * C
EPW = EPAD // NW
NCHUNK = EPW // C
EROWS = EPAD // 128
RPW = EPW // 128
RPS = N // NS

_mesh = plsc.VectorSubcoreMesh(
    core_axis_name="c", subcore_axis_name="s", num_cores=NC, num_subcores=NS)


@functools.partial(
    pl.kernel,
    out_type=jax.ShapeDtypeStruct((NC * N,), jnp.float32),
    mesh=_mesh,
    scratch_types=[
        pltpu.VMEM((K, 128), jnp.int32),
        pltpu.VMEM((K, 128), jnp.float32),
        pltpu.VMEM_SHARED((N,), jnp.float32),
    ],
)
def _deg_call(dst_hbm, ew_hbm, zeros1_hbm, out_hbm, dst_v, ew_v, deg_sh):
    c = lax.axis_index("c")
    s = lax.axis_index("s")
    wid = s * NC + c

    @pl.when(s == 0)
    def _():
        pltpu.sync_copy(zeros1_hbm, deg_sh)
    plsc.subcore_barrier()

    base_row = wid * RPW

    def chunk(i, carry):
        r0 = base_row + i * K
        pltpu.sync_copy(dst_hbm.at[pl.ds(r0, K)], dst_v)
        pltpu.sync_copy(ew_hbm.at[pl.ds(r0, K)], ew_v)
        for j in range(K):
            pltpu.sync_copy(ew_v.at[j], deg_sh.at[dst_v.at[j]], add=True)
        return carry

    lax.fori_loop(0, NCHUNK, chunk, 0)
    plsc.subcore_barrier()

    @pl.when(s == 0)
    def _():
        pltpu.sync_copy(deg_sh, out_hbm.at[pl.ds(c * N, N)])


@functools.partial(
    pl.kernel,
    out_type=jax.ShapeDtypeStruct((NC * N, H2), jnp.float32),
    mesh=_mesh,
    scratch_types=[
        pltpu.VMEM((K, 128), jnp.int32),
        pltpu.VMEM((K, 128), jnp.int32),
        pltpu.VMEM((K, 128), jnp.float32),
        pltpu.VMEM((C, H2), jnp.float32),
        pltpu.VMEM_SHARED((N, H2), jnp.float32),
        pltpu.SemaphoreType.DMA,
    ],
)
def _conv_call(t_hbm, src_hbm, dst_hbm, ew_hbm, zeros2_hbm, out_hbm,
               src_v, dst_v, ew_v, rows_v, agg_sh, sem):
    c = lax.axis_index("c")
    s = lax.axis_index("s")
    wid = s * NC + c

    pltpu.sync_copy(zeros2_hbm.at[pl.ds(s * RPS, RPS)],
                    agg_sh.at[pl.ds(s * RPS, RPS)])
    plsc.subcore_barrier()

    base_row = wid * RPW

    def chunk(i, carry):
        r0 = base_row + i * K
        pltpu.sync_copy(src_hbm.at[pl.ds(r0, K)], src_v)
        pltpu.sync_copy(dst_hbm.at[pl.ds(r0, K)], dst_v)
        pltpu.sync_copy(ew_hbm.at[pl.ds(r0, K)], ew_v)
        descs = []
        for j in range(K):
            descs.append(pltpu.async_copy(
                t_hbm.at[src_v.at[j]], rows_v.at[pl.ds(j * 128, 128)], sem))
        for d in descs:
            d.wait()
        for j in range(K):
            def scale(e, carry2):
                w = ew_v[j, e]
                rows_v[j * 128 + e, :] = rows_v[j * 128 + e, :] * w
                return carry2
            lax.fori_loop(0, 128, scale, 0)
        for j in range(K):
            pltpu.sync_copy(rows_v.at[pl.ds(j * 128, 128)],
                            agg_sh.at[dst_v.at[j]], add=True)
        return carry

    lax.fori_loop(0, NCHUNK, chunk, 0)
    plsc.subcore_barrier()

    pltpu.sync_copy(agg_sh.at[pl.ds(s * RPS, RPS)],
                    out_hbm.at[pl.ds(c * N + s * RPS, RPS)])


_R = 800
_GRID = N // _R


def _enc_body(x_ref, d0_ref, d1_ref, w1_ref, b1_ref, w2_ref, b2_ref,
              wg1_ref, t_ref, dinv_ref):
    deg = d0_ref[...] + d1_ref[...] + 1.0
    dinv = lax.rsqrt(deg)
    h = jnp.maximum(
        jnp.dot(x_ref[...], w1_ref[...], preferred_element_type=jnp.float32)
        + b1_ref[...], 0.0)
    h = jnp.maximum(
        jnp.dot(h, w2_ref[...], preferred_element_type=jnp.float32)
        + b2_ref[...], 0.0)
    t_ref[...] = jnp.dot(h, wg1_ref[...],
                         preferred_element_type=jnp.float32) * dinv
    dinv_ref[...] = dinv


def _row_spec(cols):
    return pl.BlockSpec((_R, cols), lambda i: (i, 0))


def _full_spec(r, cols):
    return pl.BlockSpec((r, cols), lambda i: (0, 0))


def _enc_call(x, d0, d1, w1, b1, w2, b2, wg1):
    return pl.pallas_call(
        _enc_body,
        grid=(_GRID,),
        in_specs=[
            _row_spec(D_IN), _row_spec(1), _row_spec(1),
            _full_spec(D_IN, H1), _full_spec(1, H1),
            _full_spec(H1, H2), _full_spec(1, H2),
            _full_spec(H2, H2),
        ],
        out_specs=[_row_spec(H2), _row_spec(1)],
        out_shape=[jax.ShapeDtypeStruct((N, H2), jnp.float32),
                   jax.ShapeDtypeStruct((N, 1), jnp.float32)],
    )(x, d0, d1, w1, b1, w2, b2, wg1)


def _mid_body(a0_ref, a1_ref, t1_ref, dinv_ref, bg1_ref, wg2_ref, t2_ref):
    agg = a0_ref[...] + a1_ref[...] + t1_ref[...]
    out1 = jnp.maximum(agg * dinv_ref[...] + bg1_ref[...], 0.0)
    t2_ref[...] = jnp.dot(out1, wg2_ref[...],
                          preferred_element_type=jnp.float32) * dinv_ref[...]


def _mid_call(a0, a1, t1, dinv, bg1, wg2):
    return pl.pallas_call(
        _mid_body,
        grid=(_GRID,),
        in_specs=[
            _row_spec(H2), _row_spec(H2), _row_spec(H2), _row_spec(1),
            _full_spec(1, H2), _full_spec(H2, H2),
        ],
        out_specs=_row_spec(H2),
        out_shape=jax.ShapeDtypeStruct((N, H2), jnp.float32),
    )(a0, a1, t1, dinv, bg1, wg2)


def _dec_body(a0_ref, a1_ref, t2_ref, dinv_ref, bg2_ref, w3_ref, b3_ref,
              w4_ref, b4_ref, y_ref):
    agg = a0_ref[...] + a1_ref[...] + t2_ref[...]
    out2 = jnp.maximum(agg * dinv_ref[...] + bg2_ref[...], 0.0)
    h = jnp.maximum(
        jnp.dot(out2, w3_ref[...], preferred_element_type=jnp.float32)
        + b3_ref[...], 0.0)
    y = jnp.dot(h, w4_ref[...], preferred_element_type=jnp.float32) + b4_ref[...]
    y_ref[...] = jax.nn.sigmoid(y)


def _dec_call(a0, a1, t2, dinv, bg2, w3, b3, w4, b4):
    return pl.pallas_call(
        _dec_body,
        grid=(_GRID,),
        in_specs=[
            _row_spec(H2), _row_spec(H2), _row_spec(H2), _row_spec(1),
            _full_spec(1, H2), _full_spec(H2, H1), _full_spec(1, H1),
            _full_spec(H1, D_IN), _full_spec(1, D_IN),
        ],
        out_specs=_row_spec(D_IN),
        out_shape=jax.ShapeDtypeStruct((N, D_IN), jnp.float32),
    )(a0, a1, t2, dinv, bg2, w3, b3, w4, b4)


def kernel(x, edge_index, edge_weight, W1, b1, W2, b2, Wg1, bg1, Wg2, bg2,
           W3, b3, W4, b4):
    pad = EPAD - E
    src = jnp.concatenate(
        [edge_index[0], jnp.zeros((pad,), jnp.int32)]).reshape(EROWS, 128)
    dst = jnp.concatenate(
        [edge_index[1], jnp.zeros((pad,), jnp.int32)]).reshape(EROWS, 128)
    ew = jnp.concatenate(
        [edge_weight, jnp.zeros((pad,), jnp.float32)]).reshape(EROWS, 128)
    zeros1 = jnp.zeros((N,), jnp.float32)
    zeros2 = jnp.zeros((N, H2), jnp.float32)

    degs = _deg_call(dst, ew, zeros1)
    d0 = degs[:N].reshape(N, 1)
    d1 = degs[N:].reshape(N, 1)

    t1, dinv = _enc_call(x, d0, d1, W1, b1.reshape(1, H1), W2,
                         b2.reshape(1, H2), Wg1)
    agg1 = _conv_call(t1, src, dst, ew, zeros2)
    t2 = _mid_call(agg1[:N], agg1[N:], t1, dinv, bg1.reshape(1, H2), Wg2)
    agg2 = _conv_call(t2, src, dst, ew, zeros2)
    return _dec_call(agg2[:N], agg2[N:], t2, dinv, bg2.reshape(1, H2), W3,
                     b3.reshape(1, H1), W4, b4.reshape(1, D_IN))


# speedup vs baseline: 36.7462x; 36.7462x over previous
import functools

import jax
import jax.numpy as jnp
from jax import lax
from jax.experimental import pallas as pl
from jax.experimental.pallas import tpu as pltpu
from jax.experimental.pallas import tpu_sc as plsc

N = 100000
E = 3200000
D_IN = 128
H1 = 64
H2 = 16

NC = 2
NS = 16
NW = NC * NS

C = 1024
K = C // 128
EPAD = 98 * NW * C
EPW = EPAD // NW
NCHUNK = EPW // C
EROWS = EPAD // 128
RPW = EPW // 128
NP = 100096
RPS = NP // NS

_mesh = plsc.VectorSubcoreMesh(
    core_axis_name="c", subcore_axis_name="s", num_cores=NC, num_subcores=NS)


@functools.partial(
    pl.kernel,
    out_type=jax.ShapeDtypeStruct((NC * NP,), jnp.float32),
    mesh=_mesh,
    scratch_types=[
        pltpu.VMEM((K, 128), jnp.int32),
        pltpu.VMEM((K, 128), jnp.float32),
        pltpu.VMEM_SHARED((NP,), jnp.float32),
    ],
)
def _deg_call(dst_hbm, ew_hbm, zeros1_hbm, out_hbm, dst_v, ew_v, deg_sh):
    c = lax.axis_index("c")
    s = lax.axis_index("s")
    wid = s * NC + c

    @pl.when(s == 0)
    def _():
        pltpu.sync_copy(zeros1_hbm, deg_sh)
    plsc.subcore_barrier()

    base_row = wid * RPW

    def chunk(i, carry):
        r0 = base_row + i * K
        pltpu.sync_copy(dst_hbm.at[pl.ds(r0, K)], dst_v)
        pltpu.sync_copy(ew_hbm.at[pl.ds(r0, K)], ew_v)
        for j in range(K):
            pltpu.sync_copy(ew_v.at[j], deg_sh.at[dst_v.at[j]], add=True)
        return carry

    lax.fori_loop(0, NCHUNK, chunk, 0)
    plsc.subcore_barrier()

    @pl.when(s == 0)
    def _():
        pltpu.sync_copy(deg_sh, out_hbm.at[pl.ds(c * NP, NP)])


@functools.partial(
    pl.kernel,
    out_type=jax.ShapeDtypeStruct((NC * NP, H2), jnp.float32),
    mesh=_mesh,
    scratch_types=[
        pltpu.VMEM((K, 128), jnp.int32),
        pltpu.VMEM((K, 128), jnp.int32),
        pltpu.VMEM((K, 128), jnp.float32),
        pltpu.VMEM((C, H2), jnp.float32),
        pltpu.VMEM_SHARED((NP, H2), jnp.float32),
        pltpu.SemaphoreType.DMA,
    ],
    compiler_params=pltpu.CompilerParams(use_tc_tiling_on_sc=False),
)
def _conv_call(t_hbm, src_hbm, dst_hbm, ew_hbm, zeros2_hbm, out_hbm,
               src_v, dst_v, ew_v, rows_v, agg_sh, sem):
    c = lax.axis_index("c")
    s = lax.axis_index("s")
    wid = s * NC + c

    pltpu.sync_copy(zeros2_hbm.at[pl.ds(s * RPS, RPS)],
                    agg_sh.at[pl.ds(s * RPS, RPS)])
    plsc.subcore_barrier()

    base_row = wid * RPW

    def chunk(i, carry):
        r0 = base_row + i * K
        pltpu.sync_copy(src_hbm.at[pl.ds(r0, K)], src_v)
        pltpu.sync_copy(dst_hbm.at[pl.ds(r0, K)], dst_v)
        pltpu.sync_copy(ew_hbm.at[pl.ds(r0, K)], ew_v)
        descs = []
        for j in range(K):
            descs.append(pltpu.async_copy(
                t_hbm.at[src_v.at[j]], rows_v.at[pl.ds(j * 128, 128)], sem))
        for d in descs:
            d.wait()
        for j in range(K):
            def scale(g, carry2):
                ew16 = ew_v[j, pl.ds(g * 16, 16)]
                base = j * 128 + g * 16
                for l in range(16):
                    rows_v[base + l, :] = rows_v[base + l, :] * ew16[l]
                return carry2
            lax.fori_loop(0, 8, scale, 0)
        for j in range(K):
            pltpu.sync_copy(rows_v.at[pl.ds(j * 128, 128)],
                            agg_sh.at[dst_v.at[j]], add=True)
        return carry

    lax.fori_loop(0, NCHUNK, chunk, 0)
    plsc.subcore_barrier()

    pltpu.sync_copy(agg_sh.at[pl.ds(s * RPS, RPS)],
                    out_hbm.at[pl.ds(c * NP + s * RPS, RPS)])


_R = 800
_GRID = N // _R


def _enc_body(x_ref, d0_ref, d1_ref, w1_ref, b1_ref, w2_ref, b2_ref,
              wg1_ref, t_ref, dinv_ref):
    deg = d0_ref[...] + d1_ref[...] + 1.0
    dinv = lax.rsqrt(deg)
    h = jnp.maximum(
        jnp.dot(x_ref[...], w1_ref[...], preferred_element_type=jnp.float32)
        + b1_ref[...], 0.0)
    h = jnp.maximum(
        jnp.dot(h, w2_ref[...], preferred_element_type=jnp.float32)
        + b2_ref[...], 0.0)
    t_ref[...] = jnp.dot(h, wg1_ref[...],
                         preferred_element_type=jnp.float32) * dinv
    dinv_ref[...] = dinv


def _row_spec(cols):
    return pl.BlockSpec((_R, cols), lambda i: (i, 0))


def _full_spec(r, cols):
    return pl.BlockSpec((r, cols), lambda i: (0, 0))


def _enc_call(x, d0, d1, w1, b1, w2, b2, wg1):
    return pl.pallas_call(
        _enc_body,
        grid=(_GRID,),
        in_specs=[
            _row_spec(D_IN), _row_spec(1), _row_spec(1),
            _full_spec(D_IN, H1), _full_spec(1, H1),
            _full_spec(H1, H2), _full_spec(1, H2),
            _full_spec(H2, H2),
        ],
        out_specs=[_row_spec(H2), _row_spec(1)],
        out_shape=[jax.ShapeDtypeStruct((N, H2), jnp.float32),
                   jax.ShapeDtypeStruct((N, 1), jnp.float32)],
    )(x, d0, d1, w1, b1, w2, b2, wg1)


def _mid_body(a0_ref, a1_ref, t1_ref, dinv_ref, bg1_ref, wg2_ref, t2_ref):
    agg = a0_ref[...] + a1_ref[...] + t1_ref[...]
    out1 = jnp.maximum(agg * dinv_ref[...] + bg1_ref[...], 0.0)
    t2_ref[...] = jnp.dot(out1, wg2_ref[...],
                          preferred_element_type=jnp.float32) * dinv_ref[...]


def _mid_call(a0, a1, t1, dinv, bg1, wg2):
    return pl.pallas_call(
        _mid_body,
        grid=(_GRID,),
        in_specs=[
            _row_spec(H2), _row_spec(H2), _row_spec(H2), _row_spec(1),
            _full_spec(1, H2), _full_spec(H2, H2),
        ],
        out_specs=_row_spec(H2),
        out_shape=jax.ShapeDtypeStruct((N, H2), jnp.float32),
    )(a0, a1, t1, dinv, bg1, wg2)


def _dec_body(a0_ref, a1_ref, t2_ref, dinv_ref, bg2_ref, w3_ref, b3_ref,
              w4_ref, b4_ref, y_ref):
    agg = a0_ref[...] + a1_ref[...] + t2_ref[...]
    out2 = jnp.maximum(agg * dinv_ref[...] + bg2_ref[...], 0.0)
    h = jnp.maximum(
        jnp.dot(out2, w3_ref[...], preferred_element_type=jnp.float32)
        + b3_ref[...], 0.0)
    y = jnp.dot(h, w4_ref[...], preferred_element_type=jnp.float32) + b4_ref[...]
    y_ref[...] = jax.nn.sigmoid(y)


def _dec_call(a0, a1, t2, dinv, bg2, w3, b3, w4, b4):
    return pl.pallas_call(
        _dec_body,
        grid=(_GRID,),
        in_specs=[
            _row_spec(H2), _row_spec(H2), _row_spec(H2), _row_spec(1),
            _full_spec(1, H2), _full_spec(H2, H1), _full_spec(1, H1),
            _full_spec(H1, D_IN), _full_spec(1, D_IN),
        ],
        out_specs=_row_spec(D_IN),
        out_shape=jax.ShapeDtypeStruct((N, D_IN), jnp.float32),
    )(a0, a1, t2, dinv, bg2, w3, b3, w4, b4)


def kernel(x, edge_index, edge_weight, W1, b1, W2, b2, Wg1, bg1, Wg2, bg2,
           W3, b3, W4, b4):
    pad = EPAD - E
    src = jnp.concatenate(
        [edge_index[0], jnp.zeros((pad,), jnp.int32)]).reshape(EROWS, 128)
    dst = jnp.concatenate(
        [edge_index[1], jnp.zeros((pad,), jnp.int32)]).reshape(EROWS, 128)
    ew = jnp.concatenate(
        [edge_weight, jnp.zeros((pad,), jnp.float32)]).reshape(EROWS, 128)
    zeros1 = jnp.zeros((NP,), jnp.float32)
    zeros2 = jnp.zeros((NP, H2), jnp.float32)

    degs = _deg_call(dst, ew, zeros1)
    d0 = degs[:N].reshape(N, 1)
    d1 = degs[NP:NP + N].reshape(N, 1)

    t1, dinv = _enc_call(x, d0, d1, W1, b1.reshape(1, H1), W2,
                         b2.reshape(1, H2), Wg1)
    agg1 = _conv_call(t1, src, dst, ew, zeros2)
    t2 = _mid_call(agg1[:N], agg1[NP:NP + N], t1, dinv, bg1.reshape(1, H2), Wg2)
    agg2 = _conv_call(t2, src, dst, ew, zeros2)
    return _dec_call(agg2[:N], agg2[NP:NP + N], t2, dinv, bg2.reshape(1, H2), W3,
                     b3.reshape(1, H1), W4, b4.reshape(1, D_IN))


# speedup vs baseline: 47.6517x; 1.2968x over previous
import functools

import jax
import jax.numpy as jnp
from jax import lax
from jax.experimental import pallas as pl
from jax.experimental.pallas import tpu as pltpu
from jax.experimental.pallas import tpu_sc as plsc

N = 100000
E = 3200000
D_IN = 128
H1 = 64
H2 = 16

NC = 2
NS = 16
NW = NC * NS

C = 512
K = C // 128
CG = C // 16
C2 = 1024
K2 = C2 // 128
EPAD = 196 * NW * C
EPW = EPAD // NW
NCHUNK = EPW // C
NCHUNK2 = EPW // C2
EROWS = EPAD // 128
RPW = EPW // 128
NP = 100096
RPS = NP // NS

_mesh = plsc.VectorSubcoreMesh(
    core_axis_name="c", subcore_axis_name="s", num_cores=NC, num_subcores=NS)


@functools.partial(
    pl.kernel,
    out_type=jax.ShapeDtypeStruct((NC * NP,), jnp.float32),
    mesh=_mesh,
    scratch_types=[
        pltpu.VMEM((K2, 128), jnp.int32),
        pltpu.VMEM((K2, 128), jnp.int32),
        pltpu.VMEM((K2, 128), jnp.float32),
        pltpu.VMEM((K2, 128), jnp.float32),
        pltpu.VMEM_SHARED((NP,), jnp.float32),
        pltpu.SemaphoreType.DMA,
        pltpu.SemaphoreType.DMA,
        pltpu.SemaphoreType.DMA,
        pltpu.SemaphoreType.DMA,
    ],
)
def _deg_call(dst_hbm, ew_hbm, zeros1_hbm, out_hbm, dst_a, dst_b, ew_a, ew_b,
              deg_sh, sia, sib, ssa, ssb):
    c = lax.axis_index("c")
    s = lax.axis_index("s")
    wid = s * NC + c

    @pl.when(s == 0)
    def _():
        pltpu.sync_copy(zeros1_hbm, deg_sh)
    plsc.subcore_barrier()

    base_row = wid * (RPW // 1)

    def pair(i2, carry):
        ra = base_row + (2 * i2) * K2
        rb = ra + K2
        ia = [pltpu.async_copy(dst_hbm.at[pl.ds(ra, K2)], dst_a, sia),
              pltpu.async_copy(ew_hbm.at[pl.ds(ra, K2)], ew_a, sia)]
        ib = [pltpu.async_copy(dst_hbm.at[pl.ds(rb, K2)], dst_b, sib),
              pltpu.async_copy(ew_hbm.at[pl.ds(rb, K2)], ew_b, sib)]
        for d in ia:
            d.wait()
        sa = [pltpu.async_copy(ew_a.at[j], deg_sh.at[dst_a.at[j]], ssa,
                               add=True) for j in range(K2)]
        for d in ib:
            d.wait()
        sb = [pltpu.async_copy(ew_b.at[j], deg_sh.at[dst_b.at[j]], ssb,
                               add=True) for j in range(K2)]
        for d in sa:
            d.wait()
        for d in sb:
            d.wait()
        return carry

    lax.fori_loop(0, NCHUNK2 // 2, pair, 0)
    plsc.subcore_barrier()

    @pl.when(s == 0)
    def _():
        pltpu.sync_copy(deg_sh, out_hbm.at[pl.ds(c * NP, NP)])


@functools.partial(
    pl.kernel,
    out_type=jax.ShapeDtypeStruct((NC * NP, H2), jnp.float32),
    mesh=_mesh,
    scratch_types=[
        pltpu.VMEM((K, 128), jnp.int32),
        pltpu.VMEM((K, 128), jnp.int32),
        pltpu.VMEM((K, 128), jnp.int32),
        pltpu.VMEM((K, 128), jnp.int32),
        pltpu.VMEM((K, 128), jnp.float32),
        pltpu.VMEM((K, 128), jnp.float32),
        pltpu.VMEM((C, H2), jnp.float32),
        pltpu.VMEM((C, H2), jnp.float32),
        pltpu.VMEM_SHARED((NP, H2), jnp.float32),
        pltpu.SemaphoreType.DMA,
        pltpu.SemaphoreType.DMA,
        pltpu.SemaphoreType.DMA,
        pltpu.SemaphoreType.DMA,
        pltpu.SemaphoreType.DMA,
        pltpu.SemaphoreType.DMA,
    ],
    compiler_params=pltpu.CompilerParams(use_tc_tiling_on_sc=False),
)
def _conv_call(t_hbm, src_hbm, dst_hbm, ew_hbm, zeros2_hbm, out_hbm,
               src_a, src_b, dst_a, dst_b, ew_a, ew_b, rows_a, rows_b,
               agg_sh, sia, sib, sga, sgb, ssa, ssb):
    c = lax.axis_index("c")
    s = lax.axis_index("s")
    wid = s * NC + c

    pltpu.sync_copy(zeros2_hbm.at[pl.ds(s * RPS, RPS)],
                    agg_sh.at[pl.ds(s * RPS, RPS)])
    plsc.subcore_barrier()

    base_row = wid * RPW

    def _scale(rows_v, ew_v):
        @plsc.parallel_loop(0, CG, 1, unroll=4)
        def _(g):
            j = g // 8
            sub = g % 8
            ew16 = ew_v[j, pl.ds(sub * 16, 16)]
            base = g * 16
            for l in range(16):
                rows_v[base + l, :] = rows_v[base + l, :] * ew16[l]

    def pair(i2, carry):
        ra = base_row + (2 * i2) * K
        rb = ra + K
        ia = [pltpu.async_copy(src_hbm.at[pl.ds(ra, K)], src_a, sia),
              pltpu.async_copy(dst_hbm.at[pl.ds(ra, K)], dst_a, sia),
              pltpu.async_copy(ew_hbm.at[pl.ds(ra, K)], ew_a, sia)]
        ib = [pltpu.async_copy(src_hbm.at[pl.ds(rb, K)], src_b, sib),
              pltpu.async_copy(dst_hbm.at[pl.ds(rb, K)], dst_b, sib),
              pltpu.async_copy(ew_hbm.at[pl.ds(rb, K)], ew_b, sib)]
        for d in ia:
            d.wait()
        ga = [pltpu.async_copy(t_hbm.at[src_a.at[j]],
                               rows_a.at[pl.ds(j * 128, 128)], sga)
              for j in range(K)]
        for d in ib:
            d.wait()
        gb = [pltpu.async_copy(t_hbm.at[src_b.at[j]],
                               rows_b.at[pl.ds(j * 128, 128)], sgb)
              for j in range(K)]
        for d in ga:
            d.wait()
        _scale(rows_a, ew_a)
        sa = [pltpu.async_copy(rows_a.at[pl.ds(j * 128, 128)],
                               agg_sh.at[dst_a.at[j]], ssa, add=True)
              for j in range(K)]
        for d in gb:
            d.wait()
        _scale(rows_b, ew_b)
        sb = [pltpu.async_copy(rows_b.at[pl.ds(j * 128, 128)],
                               agg_sh.at[dst_b.at[j]], ssb, add=True)
              for j in range(K)]
        for d in sa:
            d.wait()
        for d in sb:
            d.wait()
        return carry

    lax.fori_loop(0, NCHUNK // 2, pair, 0)
    plsc.subcore_barrier()

    pltpu.sync_copy(agg_sh.at[pl.ds(s * RPS, RPS)],
                    out_hbm.at[pl.ds(c * NP + s * RPS, RPS)])


_R = 800
_GRID = N // _R


def _enc_body(x_ref, d0_ref, d1_ref, w1_ref, b1_ref, w2_ref, b2_ref,
              wg1_ref, t_ref, dinv_ref):
    deg = d0_ref[...] + d1_ref[...] + 1.0
    dinv = lax.rsqrt(deg)
    h = jnp.maximum(
        jnp.dot(x_ref[...], w1_ref[...], preferred_element_type=jnp.float32)
        + b1_ref[...], 0.0)
    h = jnp.maximum(
        jnp.dot(h, w2_ref[...], preferred_element_type=jnp.float32)
        + b2_ref[...], 0.0)
    t_ref[...] = jnp.dot(h, wg1_ref[...],
                         preferred_element_type=jnp.float32) * dinv
    dinv_ref[...] = dinv


def _row_spec(cols):
    return pl.BlockSpec((_R, cols), lambda i: (i, 0))


def _full_spec(r, cols):
    return pl.BlockSpec((r, cols), lambda i: (0, 0))


def _enc_call(x, d0, d1, w1, b1, w2, b2, wg1):
    return pl.pallas_call(
        _enc_body,
        grid=(_GRID,),
        in_specs=[
            _row_spec(D_IN), _row_spec(1), _row_spec(1),
            _full_spec(D_IN, H1), _full_spec(1, H1),
            _full_spec(H1, H2), _full_spec(1, H2),
            _full_spec(H2, H2),
        ],
        out_specs=[_row_spec(H2), _row_spec(1)],
        out_shape=[jax.ShapeDtypeStruct((N, H2), jnp.float32),
                   jax.ShapeDtypeStruct((N, 1), jnp.float32)],
    )(x, d0, d1, w1, b1, w2, b2, wg1)


def _mid_body(a0_ref, a1_ref, t1_ref, dinv_ref, bg1_ref, wg2_ref, t2_ref):
    agg = a0_ref[...] + a1_ref[...] + t1_ref[...]
    out1 = jnp.maximum(agg * dinv_ref[...] + bg1_ref[...], 0.0)
    t2_ref[...] = jnp.dot(out1, wg2_ref[...],
                          preferred_element_type=jnp.float32) * dinv_ref[...]


def _mid_call(a0, a1, t1, dinv, bg1, wg2):
    return pl.pallas_call(
        _mid_body,
        grid=(_GRID,),
        in_specs=[
            _row_spec(H2), _row_spec(H2), _row_spec(H2), _row_spec(1),
            _full_spec(1, H2), _full_spec(H2, H2),
        ],
        out_specs=_row_spec(H2),
        out_shape=jax.ShapeDtypeStruct((N, H2), jnp.float32),
    )(a0, a1, t1, dinv, bg1, wg2)


def _dec_body(a0_ref, a1_ref, t2_ref, dinv_ref, bg2_ref, w3_ref, b3_ref,
              w4_ref, b4_ref, y_ref):
    agg = a0_ref[...] + a1_ref[...] + t2_ref[...]
    out2 = jnp.maximum(agg * dinv_ref[...] + bg2_ref[...], 0.0)
    h = jnp.maximum(
        jnp.dot(out2, w3_ref[...], preferred_element_type=jnp.float32)
        + b3_ref[...], 0.0)
    y = jnp.dot(h, w4_ref[...], preferred_element_type=jnp.float32) + b4_ref[...]
    y_ref[...] = jax.nn.sigmoid(y)


def _dec_call(a0, a1, t2, dinv, bg2, w3, b3, w4, b4):
    return pl.pallas_call(
        _dec_body,
        grid=(_GRID,),
        in_specs=[
            _row_spec(H2), _row_spec(H2), _row_spec(H2), _row_spec(1),
            _full_spec(1, H2), _full_spec(H2, H1), _full_spec(1, H1),
            _full_spec(H1, D_IN), _full_spec(1, D_IN),
        ],
        out_specs=_row_spec(D_IN),
        out_shape=jax.ShapeDtypeStruct((N, D_IN), jnp.float32),
    )(a0, a1, t2, dinv, bg2, w3, b3, w4, b4)


def kernel(x, edge_index, edge_weight, W1, b1, W2, b2, Wg1, bg1, Wg2, bg2,
           W3, b3, W4, b4):
    pad = EPAD - E
    src = jnp.concatenate(
        [edge_index[0], jnp.zeros((pad,), jnp.int32)]).reshape(EROWS, 128)
    dst = jnp.concatenate(
        [edge_index[1], jnp.zeros((pad,), jnp.int32)]).reshape(EROWS, 128)
    ew = jnp.concatenate(
        [edge_weight, jnp.zeros((pad,), jnp.float32)]).reshape(EROWS, 128)
    zeros1 = jnp.zeros((NP,), jnp.float32)
    zeros2 = jnp.zeros((NP, H2), jnp.float32)

    degs = _deg_call(dst, ew, zeros1)
    d0 = degs[:N].reshape(N, 1)
    d1 = degs[NP:NP + N].reshape(N, 1)

    t1, dinv = _enc_call(x, d0, d1, W1, b1.reshape(1, H1), W2,
                         b2.reshape(1, H2), Wg1)
    agg1 = _conv_call(t1, src, dst, ew, zeros2)
    t2 = _mid_call(agg1[:N], agg1[NP:NP + N], t1, dinv, bg1.reshape(1, H2), Wg2)
    agg2 = _conv_call(t2, src, dst, ew, zeros2)
    return _dec_call(agg2[:N], agg2[NP:NP + N], t2, dinv, bg2.reshape(1, H2), W3,
                     b3.reshape(1, H1), W4, b4.reshape(1, D_IN))


# speedup vs baseline: 62.0515x; 1.3022x over previous
import functools

import jax
import jax.numpy as jnp
from jax import lax
from jax.experimental import pallas as pl
from jax.experimental.pallas import tpu as pltpu
from jax.experimental.pallas import tpu_sc as plsc

N = 100000
E = 3200000
D_IN = 128
H1 = 64
H2 = 16

NC = 2
NS = 16
NW = NC * NS

ER = E // 128
RB = ER // NW
RX = ER % NW

C = 512
K = C // 128
CG = C // 16
PAIRS = RB // (2 * K) * 1
PROWS = PAIRS * 2 * K

C2 = 1024
K2 = C2 // 128
PAIRS2 = RB // (2 * K2)
PROWS2 = PAIRS2 * 2 * K2

NP = 102400
NPK = NP // 8
RPS = NP // NS

_mesh = plsc.VectorSubcoreMesh(
    core_axis_name="c", subcore_axis_name="s", num_cores=NC, num_subcores=NS)


def _tile_rows(wid):
    extra = (wid < RX).astype(jnp.int32)
    row0 = wid * RB + jnp.minimum(wid, RX)
    return row0, RB + extra


@functools.partial(
    pl.kernel,
    out_type=(jax.ShapeDtypeStruct((NP,), jnp.float32),
              jax.ShapeDtypeStruct((NP,), jnp.float32)),
    mesh=_mesh,
    scratch_types=[
        pltpu.VMEM((K2, 128), jnp.int32),
        pltpu.VMEM((K2, 128), jnp.int32),
        pltpu.VMEM((K2, 128), jnp.float32),
        pltpu.VMEM((K2, 128), jnp.float32),
        pltpu.VMEM_SHARED((NP,), jnp.float32),
        pltpu.SemaphoreType.DMA,
        pltpu.SemaphoreType.DMA,
        pltpu.SemaphoreType.DMA,
        pltpu.SemaphoreType.DMA,
    ],
    compiler_params=pltpu.CompilerParams(use_tc_tiling_on_sc=False),
)
def _deg_call(ei_hbm, ew_hbm, zeros1_hbm, out0_hbm, out1_hbm,
              dst_a, dst_b, ew_a, ew_b, deg_sh, sia, sib, ssa, ssb):
    c = lax.axis_index("c")
    s = lax.axis_index("s")
    wid = s * NC + c
    row0, nrows = _tile_rows(wid)

    @pl.when(s == 0)
    def _():
        pltpu.sync_copy(zeros1_hbm, deg_sh)
    plsc.subcore_barrier()

    dst_src = ei_hbm.at[1]

    def pair(i2, carry):
        ra = row0 + i2 * (2 * K2)
        rb = ra + K2
        ia = [pltpu.async_copy(dst_src.at[pl.ds(ra, K2)], dst_a, sia),
              pltpu.async_copy(ew_hbm.at[pl.ds(ra, K2)], ew_a, sia)]
        ib = [pltpu.async_copy(dst_src.at[pl.ds(rb, K2)], dst_b, sib),
              pltpu.async_copy(ew_hbm.at[pl.ds(rb, K2)], ew_b, sib)]
        for d in ia:
            d.wait()
        sa = [pltpu.async_copy(ew_a.at[j], deg_sh.at[dst_a.at[j]], ssa,
                               add=True) for j in range(K2)]
        for d in ib:
            d.wait()
        sb = [pltpu.async_copy(ew_b.at[j], deg_sh.at[dst_b.at[j]], ssb,
                               add=True) for j in range(K2)]
        for d in sa:
            d.wait()
        for d in sb:
            d.wait()
        return carry

    lax.fori_loop(0, PAIRS2, pair, 0)

    def tail(r, carry):
        row = row0 + r
        pltpu.sync_copy(dst_src.at[row], dst_a.at[0])
        pltpu.sync_copy(ew_hbm.at[row], ew_a.at[0])
        pltpu.sync_copy(ew_a.at[0], deg_sh.at[dst_a.at[0]], add=True)
        return carry

    lax.fori_loop(PROWS2, nrows, tail, 0)
    plsc.subcore_barrier()

    @pl.when((s == 0) & (c == 0))
    def _():
        pltpu.sync_copy(deg_sh, out0_hbm)

    @pl.when((s == 0) & (c == 1))
    def _():
        pltpu.sync_copy(deg_sh, out1_hbm)


@functools.partial(
    pl.kernel,
    out_type=(jax.ShapeDtypeStruct((NP, H2), jnp.float32),
              jax.ShapeDtypeStruct((NP, H2), jnp.float32)),
    mesh=_mesh,
    scratch_types=[
        pltpu.VMEM((K, 128), jnp.int32),
        pltpu.VMEM((K, 128), jnp.int32),
        pltpu.VMEM((K, 128), jnp.int32),
        pltpu.VMEM((K, 128), jnp.int32),
        pltpu.VMEM((K, 128), jnp.float32),
        pltpu.VMEM((K, 128), jnp.float32),
        pltpu.VMEM((C, H2), jnp.float32),
        pltpu.VMEM((C, H2), jnp.float32),
        pltpu.VMEM_SHARED((NP, H2), jnp.float32),
        pltpu.SemaphoreType.DMA,
        pltpu.SemaphoreType.DMA,
        pltpu.SemaphoreType.DMA,
        pltpu.SemaphoreType.DMA,
        pltpu.SemaphoreType.DMA,
        pltpu.SemaphoreType.DMA,
    ],
    compiler_params=pltpu.CompilerParams(use_tc_tiling_on_sc=False),
)
def _conv_call(t_hbm, ei_hbm, ew_hbm, zeros2_hbm, out0_hbm, out1_hbm,
               src_a, src_b, dst_a, dst_b, ew_a, ew_b, rows_a, rows_b,
               agg_sh, sia, sib, sga, sgb, ssa, ssb):
    c = lax.axis_index("c")
    s = lax.axis_index("s")
    wid = s * NC + c
    row0, nrows = _tile_rows(wid)

    pltpu.sync_copy(zeros2_hbm.at[pl.ds(s * RPS, RPS)],
                    agg_sh.at[pl.ds(s * RPS, RPS)])
    plsc.subcore_barrier()

    src_src = ei_hbm.at[0]
    dst_src = ei_hbm.at[1]

    def _scale(rows_v, ew_v):
        @plsc.parallel_loop(0, CG, 1, unroll=4)
        def _(g):
            j = g // 8
            sub = g % 8
            ew16 = ew_v[j, pl.ds(sub * 16, 16)]
            base = g * 16
            for l in range(16):
                rows_v[base + l, :] = rows_v[base + l, :] * ew16[l]

    def pair(i2, carry):
        ra = row0 + i2 * (2 * K)
        rb = ra + K
        ia = [pltpu.async_copy(src_src.at[pl.ds(ra, K)], src_a, sia),
              pltpu.async_copy(dst_src.at[pl.ds(ra, K)], dst_a, sia),
              pltpu.async_copy(ew_hbm.at[pl.ds(ra, K)], ew_a, sia)]
        ib = [pltpu.async_copy(src_src.at[pl.ds(rb, K)], src_b, sib),
              pltpu.async_copy(dst_src.at[pl.ds(rb, K)], dst_b, sib),
              pltpu.async_copy(ew_hbm.at[pl.ds(rb, K)], ew_b, sib)]
        for d in ia:
            d.wait()
        ga = [pltpu.async_copy(t_hbm.at[src_a.at[j]],
                               rows_a.at[pl.ds(j * 128, 128)], sga)
              for j in range(K)]
        for d in ib:
            d.wait()
        gb = [pltpu.async_copy(t_hbm.at[src_b.at[j]],
                               rows_b.at[pl.ds(j * 128, 128)], sgb)
              for j in range(K)]
        for d in ga:
            d.wait()
        _scale(rows_a, ew_a)
        sa = [pltpu.async_copy(rows_a.at[pl.ds(j * 128, 128)],
                               agg_sh.at[dst_a.at[j]], ssa, add=True)
              for j in range(K)]
        for d in gb:
            d.wait()
        _scale(rows_b, ew_b)
        sb = [pltpu.async_copy(rows_b.at[pl.ds(j * 128, 128)],
                               agg_sh.at[dst_b.at[j]], ssb, add=True)
              for j in range(K)]
        for d in sa:
            d.wait()
        for d in sb:
            d.wait()
        return carry

    lax.fori_loop(0, PAIRS, pair, 0)

    def tail(r, carry):
        row = row0 + r
        pltpu.sync_copy(src_src.at[row], src_a.at[0])
        pltpu.sync_copy(dst_src.at[row], dst_a.at[0])
        pltpu.sync_copy(ew_hbm.at[row], ew_a.at[0])
        pltpu.async_copy(t_hbm.at[src_a.at[0]],
                         rows_a.at[pl.ds(0, 128)], sga).wait()
        for g in range(8):
            ew16 = ew_a[0, pl.ds(g * 16, 16)]
            for l in range(16):
                e = g * 16 + l
                rows_a[e, :] = rows_a[e, :] * ew16[l]
        pltpu.sync_copy(rows_a.at[pl.ds(0, 128)],
                        agg_sh.at[dst_a.at[0]], add=True)
        return carry

    lax.fori_loop(PROWS, nrows, tail, 0)
    plsc.subcore_barrier()

    @pl.when(c == 0)
    def _():
        pltpu.sync_copy(agg_sh.at[pl.ds(s * RPS, RPS)],
                        out0_hbm.at[pl.ds(s * RPS, RPS)])

    @pl.when(c == 1)
    def _():
        pltpu.sync_copy(agg_sh.at[pl.ds(s * RPS, RPS)],
                        out1_hbm.at[pl.ds(s * RPS, RPS)])


_R = 2000
_GRID = N // _R


def _row_spec(cols):
    return pl.BlockSpec((_R, cols), lambda i: (i, 0))


def _full_spec(r, cols):
    return pl.BlockSpec((r, cols), lambda i: (0, 0))


def _dinv(d0_ref, d1_ref):
    return lax.rsqrt(d0_ref[...] + d1_ref[...] + 1.0)


def _enc_body(x_ref, d0_ref, d1_ref, w1_ref, b1_ref, w2_ref, b2_ref,
              wg1_ref, t_ref):
    h = jnp.maximum(
        jnp.dot(x_ref[...], w1_ref[...], preferred_element_type=jnp.float32)
        + b1_ref[...], 0.0)
    h = jnp.maximum(
        jnp.dot(h, w2_ref[...], preferred_element_type=jnp.float32)
        + b2_ref[...], 0.0)
    t = jnp.dot(h, wg1_ref[...], preferred_element_type=jnp.float32)
    t_ref[...] = t * _dinv(d0_ref, d1_ref)


def _enc_call(x, deg0, deg1, w1, b1, w2, b2, wg1):
    return pl.pallas_call(
        _enc_body,
        grid=(_GRID,),
        in_specs=[
            _row_spec(D_IN), _row_spec(1), _row_spec(1),
            _full_spec(D_IN, H1), _full_spec(1, H1),
            _full_spec(H1, H2), _full_spec(1, H2),
            _full_spec(H2, H2),
        ],
        out_specs=_row_spec(H2),
        out_shape=jax.ShapeDtypeStruct((N, H2), jnp.float32),
    )(x, deg0, deg1, w1, b1, w2, b2, wg1)


def _mid_body(a0_ref, a1_ref, t1_ref, d0_ref, d1_ref, bg1_ref, wg2_ref,
              t2_ref):
    dinv = _dinv(d0_ref, d1_ref)
    agg = a0_ref[...] + a1_ref[...] + t1_ref[...]
    m = jnp.maximum(agg * dinv + bg1_ref[...], 0.0)
    u = jnp.dot(m, wg2_ref[...], preferred_element_type=jnp.float32)
    t2_ref[...] = u * dinv


def _mid_call(a0, a1, t1, deg0, deg1, bg1, wg2):
    return pl.pallas_call(
        _mid_body,
        grid=(_GRID,),
        in_specs=[
            _row_spec(H2), _row_spec(H2), _row_spec(H2),
            _row_spec(1), _row_spec(1),
            _full_spec(1, H2), _full_spec(H2, H2),
        ],
        out_specs=_row_spec(H2),
        out_shape=jax.ShapeDtypeStruct((N, H2), jnp.float32),
    )(a0, a1, t1, deg0, deg1, bg1, wg2)


def _dec_body(a0_ref, a1_ref, t2_ref, d0_ref, d1_ref, bg2_ref, w3_ref,
              b3_ref, w4_ref, b4_ref, y_ref):
    dinv = _dinv(d0_ref, d1_ref)
    agg = a0_ref[...] + a1_ref[...] + t2_ref[...]
    m = jnp.maximum(agg * dinv + bg2_ref[...], 0.0)
    h = jnp.maximum(
        jnp.dot(m, w3_ref[...],
                preferred_element_type=jnp.float32) + b3_ref[...], 0.0)
    y = jnp.dot(h, w4_ref[...], preferred_element_type=jnp.float32) + b4_ref[...]
    y_ref[...] = jax.nn.sigmoid(y)


def _dec_call(a0, a1, t2, deg0, deg1, bg2, w3, b3, w4, b4):
    return pl.pallas_call(
        _dec_body,
        grid=(_GRID,),
        in_specs=[
            _row_spec(H2), _row_spec(H2), _row_spec(H2),
            _row_spec(1), _row_spec(1),
            _full_spec(1, H2),
            _full_spec(H2, H1), _full_spec(1, H1),
            _full_spec(H1, D_IN), _full_spec(1, D_IN),
        ],
        out_specs=_row_spec(D_IN),
        out_shape=jax.ShapeDtypeStruct((N, D_IN), jnp.float32),
    )(a0, a1, t2, deg0, deg1, bg2, w3, b3, w4, b4)


def kernel(x, edge_index, edge_weight, W1, b1, W2, b2, Wg1, bg1, Wg2, bg2,
           W3, b3, W4, b4):
    ei3 = edge_index.reshape(2, ER, 128)
    ew2 = edge_weight.reshape(ER, 128)
    zeros1 = jnp.zeros((NP,), jnp.float32)
    zeros2 = jnp.zeros((NP, H2), jnp.float32)

    deg0, deg1 = _deg_call(ei3, ew2, zeros1)
    d0c = deg0.reshape(NP, 1)
    d1c = deg1.reshape(NP, 1)

    t1 = _enc_call(x, d0c, d1c, W1, b1.reshape(1, H1), W2, b2.reshape(1, H2),
                   Wg1)
    a10, a11 = _conv_call(t1, ei3, ew2, zeros2)
    t2 = _mid_call(a10, a11, t1, d0c, d1c, bg1.reshape(1, H2), Wg2)
    a20, a21 = _conv_call(t2, ei3, ew2, zeros2)
    return _dec_call(a20, a21, t2, d0c, d1c, bg2.reshape(1, H2), W3,
                     b3.reshape(1, H1), W4, b4.reshape(1, D_IN))


# speedup vs baseline: 70.7278x; 1.1398x over previous
import functools

import jax
import jax.numpy as jnp
from jax import lax
from jax.experimental import pallas as pl
from jax.experimental.pallas import tpu as pltpu
from jax.experimental.pallas import tpu_sc as plsc

N = 100000
E = 3200000
D_IN = 128
H1 = 64
H2 = 16

NC = 2
NS = 16
NW = NC * NS

ER = E // 128
RB = ER // NW
RX = ER % NW

C = 512
K = C // 128
CG = C // 16
PAIRS = RB // (2 * K) * 1
PROWS = PAIRS * 2 * K

C2 = 1024
K2 = C2 // 128
PAIRS2 = RB // (2 * K2)
PROWS2 = PAIRS2 * 2 * K2

NP = 102400
NPK = NP // 8
RPS = NP // NS

_mesh = plsc.VectorSubcoreMesh(
    core_axis_name="c", subcore_axis_name="s", num_cores=NC, num_subcores=NS)


def _tile_rows(wid):
    extra = (wid < RX).astype(jnp.int32)
    row0 = wid * RB + jnp.minimum(wid, RX)
    return row0, RB + extra


@functools.partial(
    pl.kernel,
    out_type=(jax.ShapeDtypeStruct((NP,), jnp.float32),
              jax.ShapeDtypeStruct((NP,), jnp.float32)),
    mesh=_mesh,
    scratch_types=[
        pltpu.VMEM((K2, 128), jnp.int32),
        pltpu.VMEM((K2, 128), jnp.int32),
        pltpu.VMEM((K2, 128), jnp.float32),
        pltpu.VMEM((K2, 128), jnp.float32),
        pltpu.VMEM_SHARED((NP,), jnp.float32),
        pltpu.SemaphoreType.DMA,
        pltpu.SemaphoreType.DMA,
        pltpu.SemaphoreType.DMA,
        pltpu.SemaphoreType.DMA,
    ],
    compiler_params=pltpu.CompilerParams(use_tc_tiling_on_sc=False),
)
def _deg_call(ei_hbm, ew_hbm, zeros1_hbm, out0_hbm, out1_hbm,
              dst_a, dst_b, ew_a, ew_b, deg_sh, sia, sib, ssa, ssb):
    c = lax.axis_index("c")
    s = lax.axis_index("s")
    wid = s * NC + c
    row0, nrows = _tile_rows(wid)

    @pl.when(s == 0)
    def _():
        pltpu.sync_copy(zeros1_hbm, deg_sh)
    plsc.subcore_barrier()

    dst_src = ei_hbm.at[1]

    def pair(i2, carry):
        ra = row0 + i2 * (2 * K2)
        rb = ra + K2
        ia = [pltpu.async_copy(dst_src.at[pl.ds(ra, K2)], dst_a, sia),
              pltpu.async_copy(ew_hbm.at[pl.ds(ra, K2)], ew_a, sia)]
        ib = [pltpu.async_copy(dst_src.at[pl.ds(rb, K2)], dst_b, sib),
              pltpu.async_copy(ew_hbm.at[pl.ds(rb, K2)], ew_b, sib)]
        for d in ia:
            d.wait()
        sa = [pltpu.async_copy(ew_a.at[j], deg_sh.at[dst_a.at[j]], ssa,
                               add=True) for j in range(K2)]
        for d in ib:
            d.wait()
        sb = [pltpu.async_copy(ew_b.at[j], deg_sh.at[dst_b.at[j]], ssb,
                               add=True) for j in range(K2)]
        for d in sa:
            d.wait()
        for d in sb:
            d.wait()
        return carry

    lax.fori_loop(0, PAIRS2, pair, 0)

    def tail(r, carry):
        row = row0 + r
        pltpu.sync_copy(dst_src.at[row], dst_a.at[0])
        pltpu.sync_copy(ew_hbm.at[row], ew_a.at[0])
        pltpu.sync_copy(ew_a.at[0], deg_sh.at[dst_a.at[0]], add=True)
        return carry

    lax.fori_loop(PROWS2, nrows, tail, 0)
    plsc.subcore_barrier()

    @pl.when((s == 0) & (c == 0))
    def _():
        pltpu.sync_copy(deg_sh, out0_hbm)

    @pl.when((s == 0) & (c == 1))
    def _():
        pltpu.sync_copy(deg_sh, out1_hbm)


@functools.partial(
    pl.kernel,
    out_type=(jax.ShapeDtypeStruct((NP, H2), jnp.float32),
              jax.ShapeDtypeStruct((NP, H2), jnp.float32)),
    mesh=_mesh,
    scratch_types=[
        pltpu.VMEM((K, 128), jnp.int32),
        pltpu.VMEM((K, 128), jnp.int32),
        pltpu.VMEM((K, 128), jnp.int32),
        pltpu.VMEM((K, 128), jnp.int32),
        pltpu.VMEM((K, 128), jnp.float32),
        pltpu.VMEM((K, 128), jnp.float32),
        pltpu.VMEM((C, H2), jnp.float32),
        pltpu.VMEM((C, H2), jnp.float32),
        pltpu.VMEM_SHARED((NP, H2), jnp.float32),
        pltpu.SemaphoreType.DMA,
        pltpu.SemaphoreType.DMA,
        pltpu.SemaphoreType.DMA,
        pltpu.SemaphoreType.DMA,
        pltpu.SemaphoreType.DMA,
        pltpu.SemaphoreType.DMA,
    ],
    compiler_params=pltpu.CompilerParams(use_tc_tiling_on_sc=False),
)
def _conv_call(t_hbm, ei_hbm, ew_hbm, zeros2_hbm, out0_hbm, out1_hbm,
               src_a, src_b, dst_a, dst_b, ew_a, ew_b, rows_a, rows_b,
               agg_sh, sia, sib, sga, sgb, ssa, ssb):
    c = lax.axis_index("c")
    s = lax.axis_index("s")
    wid = s * NC + c
    row0, nrows = _tile_rows(wid)

    pltpu.sync_copy(zeros2_hbm.at[pl.ds(s * RPS, RPS)],
                    agg_sh.at[pl.ds(s * RPS, RPS)])
    plsc.subcore_barrier()

    src_src = ei_hbm.at[0]
    dst_src = ei_hbm.at[1]

    def _scale(rows_v, ew_v):
        @plsc.parallel_loop(0, CG, 1, unroll=4)
        def _(g):
            j = g // 8
            sub = g % 8
            ew16 = ew_v[j, pl.ds(sub * 16, 16)]
            base = g * 16
            for l in range(16):
                rows_v[base + l, :] = rows_v[base + l, :] * ew16[l]

    def pair(i2, carry):
        ra = row0 + i2 * (2 * K)
        rb = ra + K
        ia = [pltpu.async_copy(src_src.at[pl.ds(ra, K)], src_a, sia),
              pltpu.async_copy(dst_src.at[pl.ds(ra, K)], dst_a, sia),
              pltpu.async_copy(ew_hbm.at[pl.ds(ra, K)], ew_a, sia)]
        ib = [pltpu.async_copy(src_src.at[pl.ds(rb, K)], src_b, sib),
              pltpu.async_copy(dst_src.at[pl.ds(rb, K)], dst_b, sib),
              pltpu.async_copy(ew_hbm.at[pl.ds(rb, K)], ew_b, sib)]
        for d in ia:
            d.wait()
        ga = [pltpu.async_copy(t_hbm.at[src_a.at[j]],
                               rows_a.at[pl.ds(j * 128, 128)], sga)
              for j in range(K)]
        for d in ib:
            d.wait()
        gb = [pltpu.async_copy(t_hbm.at[src_b.at[j]],
                               rows_b.at[pl.ds(j * 128, 128)], sgb)
              for j in range(K)]
        for d in ga:
            d.wait()
        _scale(rows_a, ew_a)
        sa = [pltpu.async_copy(rows_a.at[pl.ds(j * 128, 128)],
                               agg_sh.at[dst_a.at[j]], ssa, add=True)
              for j in range(K)]
        for d in gb:
            d.wait()
        _scale(rows_b, ew_b)
        sb = [pltpu.async_copy(rows_b.at[pl.ds(j * 128, 128)],
                               agg_sh.at[dst_b.at[j]], ssb, add=True)
              for j in range(K)]
        for d in sa:
            d.wait()
        for d in sb:
            d.wait()
        return carry

    lax.fori_loop(0, PAIRS, pair, 0)

    def tail(r, carry):
        row = row0 + r
        pltpu.sync_copy(src_src.at[row], src_a.at[0])
        pltpu.sync_copy(dst_src.at[row], dst_a.at[0])
        pltpu.sync_copy(ew_hbm.at[row], ew_a.at[0])
        pltpu.async_copy(t_hbm.at[src_a.at[0]],
                         rows_a.at[pl.ds(0, 128)], sga).wait()
        for g in range(8):
            ew16 = ew_a[0, pl.ds(g * 16, 16)]
            for l in range(16):
                e = g * 16 + l
                rows_a[e, :] = rows_a[e, :] * ew16[l]
        pltpu.sync_copy(rows_a.at[pl.ds(0, 128)],
                        agg_sh.at[dst_a.at[0]], add=True)
        return carry

    lax.fori_loop(PROWS, nrows, tail, 0)
    plsc.subcore_barrier()

    @pl.when(c == 0)
    def _():
        pltpu.sync_copy(agg_sh.at[pl.ds(s * RPS, RPS)],
                        out0_hbm.at[pl.ds(s * RPS, RPS)])

    @pl.when(c == 1)
    def _():
        pltpu.sync_copy(agg_sh.at[pl.ds(s * RPS, RPS)],
                        out1_hbm.at[pl.ds(s * RPS, RPS)])


_R = 2000
_GRID = N // _R
_RP = 256


def _row_spec(cols):
    return pl.BlockSpec((_R, cols), lambda i: (i, 0))


def _pk_spec():
    return pl.BlockSpec((_RP, 128), lambda i: (i, 0))


def _full_spec(r, cols):
    return pl.BlockSpec((r, cols), lambda i: (0, 0))


def _enc_body(x_ref, d16_ref, w1_ref, b1_ref, w2_ref, b2_ref,
              wg1_ref, t_ref):
    h = jnp.maximum(
        jnp.dot(x_ref[...], w1_ref[...], preferred_element_type=jnp.float32)
        + b1_ref[...], 0.0)
    h = jnp.maximum(
        jnp.dot(h, w2_ref[...], preferred_element_type=jnp.float32)
        + b2_ref[...], 0.0)
    t = jnp.dot(h, wg1_ref[...], preferred_element_type=jnp.float32)
    t_ref[...] = t * lax.rsqrt(d16_ref[...] + 1.0)


def _enc_call(x, deg16, w1, b1, w2, b2, wg1):
    return pl.pallas_call(
        _enc_body,
        grid=(_GRID,),
        in_specs=[
            _row_spec(D_IN), _row_spec(H2),
            _full_spec(D_IN, H1), _full_spec(1, H1),
            _full_spec(H1, H2), _full_spec(1, H2),
            _full_spec(H2, H2),
        ],
        out_specs=_row_spec(H2),
        out_shape=jax.ShapeDtypeStruct((NP, H2), jnp.float32),
    )(x, deg16, w1, b1, w2, b2, wg1)


def _mid_body(a0_ref, a1_ref, t1_ref, dp_ref, bg1p_ref, wg2k_ref, t2_ref):
    dinv = lax.rsqrt(dp_ref[...] + 1.0)
    agg = a0_ref[...] + a1_ref[...] + t1_ref[...]
    m = jnp.maximum(agg * dinv + bg1p_ref[...], 0.0)
    u = jnp.dot(m, wg2k_ref[...], preferred_element_type=jnp.float32)
    t2_ref[...] = u * dinv


def _mid_call(a0p, a1p, t1p, degp, bg1p, wg2k):
    return pl.pallas_call(
        _mid_body,
        grid=(_GRID,),
        in_specs=[
            _pk_spec(), _pk_spec(), _pk_spec(), _pk_spec(),
            _full_spec(1, 128), _full_spec(128, 128),
        ],
        out_specs=_pk_spec(),
        out_shape=jax.ShapeDtypeStruct((NPK, 128), jnp.float32),
    )(a0p, a1p, t1p, degp, bg1p, wg2k)


def _dec_body(a0_ref, a1_ref, t2_ref, d16_ref, bg2_ref, w3_ref,
              b3_ref, w4_ref, b4_ref, y_ref):
    dinv = lax.rsqrt(d16_ref[...] + 1.0)
    agg = a0_ref[...] + a1_ref[...] + t2_ref[...]
    m = jnp.maximum(agg * dinv + bg2_ref[...], 0.0)
    h = jnp.maximum(
        jnp.dot(m, w3_ref[...],
                preferred_element_type=jnp.float32) + b3_ref[...], 0.0)
    y = jnp.dot(h, w4_ref[...], preferred_element_type=jnp.float32) + b4_ref[...]
    y_ref[...] = jax.nn.sigmoid(y)


def _dec_call(a0, a1, t2, deg16, bg2, w3, b3, w4, b4):
    return pl.pallas_call(
        _dec_body,
        grid=(_GRID,),
        in_specs=[
            _row_spec(H2), _row_spec(H2), _row_spec(H2),
            _row_spec(H2),
            _full_spec(1, H2),
            _full_spec(H2, H1), _full_spec(1, H1),
            _full_spec(H1, D_IN), _full_spec(1, D_IN),
        ],
        out_specs=_row_spec(D_IN),
        out_shape=jax.ShapeDtypeStruct((N, D_IN), jnp.float32),
    )(a0, a1, t2, deg16, bg2, w3, b3, w4, b4)


def kernel(x, edge_index, edge_weight, W1, b1, W2, b2, Wg1, bg1, Wg2, bg2,
           W3, b3, W4, b4):
    ei3 = edge_index.reshape(2, ER, 128)
    ew2 = edge_weight.reshape(ER, 128)
    zeros1 = jnp.zeros((NP,), jnp.float32)
    zeros2 = jnp.zeros((NP, H2), jnp.float32)

    deg0, deg1 = _deg_call(ei3, ew2, zeros1)
    deg = deg0 + deg1
    deg16 = jnp.broadcast_to(deg.reshape(NP, 1), (NP, H2))
    degp = jnp.broadcast_to(deg.reshape(NPK, 8, 1),
                            (NPK, 8, H2)).reshape(NPK, 128)

    t1 = _enc_call(x, deg16, W1, b1.reshape(1, H1), W2, b2.reshape(1, H2),
                   Wg1)
    a10, a11 = _conv_call(t1, ei3, ew2, zeros2)
    t2p = _mid_call(a10.reshape(NPK, 128), a11.reshape(NPK, 128),
                    t1.reshape(NPK, 128), degp,
                    jnp.tile(bg1, 8).reshape(1, 128),
                    jnp.kron(jnp.eye(8, dtype=jnp.float32), Wg2))
    t2 = t2p.reshape(NP, H2)
    a20, a21 = _conv_call(t2, ei3, ew2, zeros2)
    return _dec_call(a20, a21, t2, deg16, bg2.reshape(1, H2), W3,
                     b3.reshape(1, H1), W4, b4.reshape(1, D_IN))


# speedup vs baseline: 72.4985x; 1.0250x over previous
import functools

import jax
import jax.numpy as jnp
from jax import lax
from jax.experimental import pallas as pl
from jax.experimental.pallas import tpu as pltpu
from jax.experimental.pallas import tpu_sc as plsc

N = 100000
E = 3200000
D_IN = 128
H1 = 64
H2 = 16

NC = 2
NS = 16
NW = NC * NS

ER = E // 128
RB = ER // NW
RX = ER % NW

C = 512
K = C // 128
CG = C // 16
PAIRS = RB // (2 * K) * 1
PROWS = PAIRS * 2 * K

C2 = 1024
K2 = C2 // 128
PAIRS2 = RB // (2 * K2)
PROWS2 = PAIRS2 * 2 * K2

NP = 102400
NPK = NP // 8
RPS = NP // NS

_mesh = plsc.VectorSubcoreMesh(
    core_axis_name="c", subcore_axis_name="s", num_cores=NC, num_subcores=NS)


def _tile_rows(wid):
    extra = (wid < RX).astype(jnp.int32)
    row0 = wid * RB + jnp.minimum(wid, RX)
    return row0, RB + extra


@functools.partial(
    pl.kernel,
    out_type=(jax.ShapeDtypeStruct((NP,), jnp.float32),
              jax.ShapeDtypeStruct((NP,), jnp.float32)),
    mesh=_mesh,
    scratch_types=[
        pltpu.VMEM((K2, 128), jnp.int32),
        pltpu.VMEM((K2, 128), jnp.int32),
        pltpu.VMEM((K2, 128), jnp.float32),
        pltpu.VMEM((K2, 128), jnp.float32),
        pltpu.VMEM_SHARED((NP,), jnp.float32),
        pltpu.SemaphoreType.DMA,
        pltpu.SemaphoreType.DMA,
        pltpu.SemaphoreType.DMA,
        pltpu.SemaphoreType.DMA,
    ],
    compiler_params=pltpu.CompilerParams(use_tc_tiling_on_sc=False),
)
def _deg_call(ei_hbm, ew_hbm, zeros1_hbm, out0_hbm, out1_hbm,
              dst_a, dst_b, ew_a, ew_b, deg_sh, sia, sib, ssa, ssb):
    c = lax.axis_index("c")
    s = lax.axis_index("s")
    wid = s * NC + c
    row0, nrows = _tile_rows(wid)

    @pl.when(s == 0)
    def _():
        pltpu.sync_copy(zeros1_hbm, deg_sh)
    plsc.subcore_barrier()

    dst_src = ei_hbm.at[1]

    def pair(i2, carry):
        ra = row0 + i2 * (2 * K2)
        rb = ra + K2
        ia = [pltpu.async_copy(dst_src.at[pl.ds(ra, K2)], dst_a, sia),
              pltpu.async_copy(ew_hbm.at[pl.ds(ra, K2)], ew_a, sia)]
        ib = [pltpu.async_copy(dst_src.at[pl.ds(rb, K2)], dst_b, sib),
              pltpu.async_copy(ew_hbm.at[pl.ds(rb, K2)], ew_b, sib)]
        for d in ia:
            d.wait()
        sa = [pltpu.async_copy(ew_a.at[j], deg_sh.at[dst_a.at[j]], ssa,
                               add=True) for j in range(K2)]
        for d in ib:
            d.wait()
        sb = [pltpu.async_copy(ew_b.at[j], deg_sh.at[dst_b.at[j]], ssb,
                               add=True) for j in range(K2)]
        for d in sa:
            d.wait()
        for d in sb:
            d.wait()
        return carry

    lax.fori_loop(0, PAIRS2, pair, 0)

    def tail(r, carry):
        row = row0 + r
        pltpu.sync_copy(dst_src.at[row], dst_a.at[0])
        pltpu.sync_copy(ew_hbm.at[row], ew_a.at[0])
        pltpu.sync_copy(ew_a.at[0], deg_sh.at[dst_a.at[0]], add=True)
        return carry

    lax.fori_loop(PROWS2, nrows, tail, 0)
    plsc.subcore_barrier()

    @pl.when((s == 0) & (c == 0))
    def _():
        pltpu.sync_copy(deg_sh, out0_hbm)

    @pl.when((s == 0) & (c == 1))
    def _():
        pltpu.sync_copy(deg_sh, out1_hbm)


@functools.partial(
    pl.kernel,
    out_type=(jax.ShapeDtypeStruct((NP, H2), jnp.float32),
              jax.ShapeDtypeStruct((NP, H2), jnp.float32)),
    mesh=_mesh,
    scratch_types=[
        pltpu.VMEM((K, 128), jnp.int32),
        pltpu.VMEM((K, 128), jnp.int32),
        pltpu.VMEM((K, 128), jnp.int32),
        pltpu.VMEM((K, 128), jnp.int32),
        pltpu.VMEM((K, 128), jnp.float32),
        pltpu.VMEM((K, 128), jnp.float32),
        pltpu.VMEM((C, H2), jnp.float32),
        pltpu.VMEM((C, H2), jnp.float32),
        pltpu.VMEM_SHARED((NP, H2), jnp.float32),
        pltpu.SemaphoreType.DMA,
        pltpu.SemaphoreType.DMA,
        pltpu.SemaphoreType.DMA,
        pltpu.SemaphoreType.DMA,
        pltpu.SemaphoreType.DMA,
        pltpu.SemaphoreType.DMA,
    ],
    compiler_params=pltpu.CompilerParams(use_tc_tiling_on_sc=False),
)
def _conv_call(t_hbm, ei_hbm, ew_hbm, zeros2_hbm, out0_hbm, out1_hbm,
               src_a, src_b, dst_a, dst_b, ew_a, ew_b, rows_a, rows_b,
               agg_sh, sia, sib, sga, sgb, ssa, ssb):
    c = lax.axis_index("c")
    s = lax.axis_index("s")
    wid = s * NC + c
    row0, nrows = _tile_rows(wid)

    pltpu.sync_copy(zeros2_hbm.at[pl.ds(s * RPS, RPS)],
                    agg_sh.at[pl.ds(s * RPS, RPS)])
    plsc.subcore_barrier()

    src_src = ei_hbm.at[0]
    dst_src = ei_hbm.at[1]

    def _scale(rows_v, ew_v):
        @plsc.parallel_loop(0, CG, 1, unroll=4)
        def _(g):
            j = g // 8
            sub = g % 8
            ew16 = ew_v[j, pl.ds(sub * 16, 16)]
            base = g * 16
            for l in range(16):
                rows_v[base + l, :] = rows_v[base + l, :] * ew16[l]

    def pair(i2, carry):
        ra = row0 + i2 * (2 * K)
        rb = ra + K
        ia = [pltpu.async_copy(src_src.at[pl.ds(ra, K)], src_a, sia),
              pltpu.async_copy(dst_src.at[pl.ds(ra, K)], dst_a, sia),
              pltpu.async_copy(ew_hbm.at[pl.ds(ra, K)], ew_a, sia)]
        ib = [pltpu.async_copy(src_src.at[pl.ds(rb, K)], src_b, sib),
              pltpu.async_copy(dst_src.at[pl.ds(rb, K)], dst_b, sib),
              pltpu.async_copy(ew_hbm.at[pl.ds(rb, K)], ew_b, sib)]
        for d in ia:
            d.wait()
        ga = [pltpu.async_copy(t_hbm.at[src_a.at[j]],
                               rows_a.at[pl.ds(j * 128, 128)], sga)
              for j in range(K)]
        for d in ib:
            d.wait()
        gb = [pltpu.async_copy(t_hbm.at[src_b.at[j]],
                               rows_b.at[pl.ds(j * 128, 128)], sgb)
              for j in range(K)]
        for d in ga:
            d.wait()
        _scale(rows_a, ew_a)
        sa = [pltpu.async_copy(rows_a.at[pl.ds(j * 128, 128)],
                               agg_sh.at[dst_a.at[j]], ssa, add=True)
              for j in range(K)]
        for d in gb:
            d.wait()
        _scale(rows_b, ew_b)
        sb = [pltpu.async_copy(rows_b.at[pl.ds(j * 128, 128)],
                               agg_sh.at[dst_b.at[j]], ssb, add=True)
              for j in range(K)]
        for d in sa:
            d.wait()
        for d in sb:
            d.wait()
        return carry

    lax.fori_loop(0, PAIRS, pair, 0)

    def tail(r, carry):
        row = row0 + r
        pltpu.sync_copy(src_src.at[row], src_a.at[0])
        pltpu.sync_copy(dst_src.at[row], dst_a.at[0])
        pltpu.sync_copy(ew_hbm.at[row], ew_a.at[0])
        pltpu.async_copy(t_hbm.at[src_a.at[0]],
                         rows_a.at[pl.ds(0, 128)], sga).wait()
        for g in range(8):
            ew16 = ew_a[0, pl.ds(g * 16, 16)]
            for l in range(16):
                e = g * 16 + l
                rows_a[e, :] = rows_a[e, :] * ew16[l]
        pltpu.sync_copy(rows_a.at[pl.ds(0, 128)],
                        agg_sh.at[dst_a.at[0]], add=True)
        return carry

    lax.fori_loop(PROWS, nrows, tail, 0)
    plsc.subcore_barrier()

    @pl.when(c == 0)
    def _():
        pltpu.sync_copy(agg_sh.at[pl.ds(s * RPS, RPS)],
                        out0_hbm.at[pl.ds(s * RPS, RPS)])

    @pl.when(c == 1)
    def _():
        pltpu.sync_copy(agg_sh.at[pl.ds(s * RPS, RPS)],
                        out1_hbm.at[pl.ds(s * RPS, RPS)])


_R = 2000
_GRID = N // _R
_RP = 256


def _row_spec(cols):
    return pl.BlockSpec((_R, cols), lambda i: (i, 0))


def _pk_spec():
    return pl.BlockSpec((_RP, 128), lambda i: (i, 0))


def _full_spec(r, cols):
    return pl.BlockSpec((r, cols), lambda i: (0, 0))


def _enc_body(x_ref, d16_ref, w1_ref, b1_ref, w2_ref, b2_ref,
              wg1_ref, t_ref):
    h = jnp.maximum(
        jnp.dot(x_ref[...], w1_ref[...], preferred_element_type=jnp.float32)
        + b1_ref[...], 0.0)
    h = jnp.maximum(
        jnp.dot(h, w2_ref[...], preferred_element_type=jnp.float32)
        + b2_ref[...], 0.0)
    t = jnp.dot(h, wg1_ref[...], preferred_element_type=jnp.float32)
    t_ref[...] = t * lax.rsqrt(d16_ref[...] + 1.0)


def _enc_call(x, deg16, w1, b1, w2, b2, wg1):
    return pl.pallas_call(
        _enc_body,
        grid=(_GRID,),
        in_specs=[
            _row_spec(D_IN), _row_spec(H2),
            _full_spec(D_IN, H1), _full_spec(1, H1),
            _full_spec(H1, H2), _full_spec(1, H2),
            _full_spec(H2, H2),
        ],
        out_specs=_row_spec(H2),
        out_shape=jax.ShapeDtypeStruct((NP, H2), jnp.float32),
    )(x, deg16, w1, b1, w2, b2, wg1)


def _mid_body(a0_ref, a1_ref, t1_ref, dp_ref, bg1p_ref, wg2k_ref, t2_ref):
    dinv = lax.rsqrt(dp_ref[...] + 1.0)
    agg = a0_ref[...] + a1_ref[...] + t1_ref[...]
    m = jnp.maximum(agg * dinv + bg1p_ref[...], 0.0)
    u = jnp.dot(m, wg2k_ref[...], preferred_element_type=jnp.float32)
    t2_ref[...] = u * dinv


def _mid_call(a0p, a1p, t1p, degp, bg1p, wg2k):
    return pl.pallas_call(
        _mid_body,
        grid=(_GRID,),
        in_specs=[
            _pk_spec(), _pk_spec(), _pk_spec(), _pk_spec(),
            _full_spec(1, 128), _full_spec(128, 128),
        ],
        out_specs=_pk_spec(),
        out_shape=jax.ShapeDtypeStruct((NPK, 128), jnp.float32),
    )(a0p, a1p, t1p, degp, bg1p, wg2k)


def _mid2_body(a0_ref, a1_ref, t2_ref, dp_ref, bg2p_ref, m_ref):
    dinv = lax.rsqrt(dp_ref[...] + 1.0)
    agg = a0_ref[...] + a1_ref[...] + t2_ref[...]
    m_ref[...] = jnp.maximum(agg * dinv + bg2p_ref[...], 0.0)


def _mid2_call(a0p, a1p, t2p, degp, bg2p):
    return pl.pallas_call(
        _mid2_body,
        grid=(_GRID,),
        in_specs=[
            _pk_spec(), _pk_spec(), _pk_spec(), _pk_spec(),
            _full_spec(1, 128),
        ],
        out_specs=_pk_spec(),
        out_shape=jax.ShapeDtypeStruct((NPK, 128), jnp.float32),
    )(a0p, a1p, t2p, degp, bg2p)


def _dec_body(m_ref, w3_ref, b3_ref, w4_ref, b4_ref, y_ref):
    h = jnp.maximum(
        jnp.dot(m_ref[...], w3_ref[...],
                preferred_element_type=jnp.float32) + b3_ref[...], 0.0)
    y = jnp.dot(h, w4_ref[...], preferred_element_type=jnp.float32) + b4_ref[...]
    y_ref[...] = jax.nn.sigmoid(y)


def _dec_call(m, w3, b3, w4, b4):
    return pl.pallas_call(
        _dec_body,
        grid=(_GRID,),
        in_specs=[
            _row_spec(H2),
            _full_spec(H2, H1), _full_spec(1, H1),
            _full_spec(H1, D_IN), _full_spec(1, D_IN),
        ],
        out_specs=_row_spec(D_IN),
        out_shape=jax.ShapeDtypeStruct((N, D_IN), jnp.float32),
    )(m, w3, b3, w4, b4)


def kernel(x, edge_index, edge_weight, W1, b1, W2, b2, Wg1, bg1, Wg2, bg2,
           W3, b3, W4, b4):
    ei3 = edge_index.reshape(2, ER, 128)
    ew2 = edge_weight.reshape(ER, 128)
    zeros1 = jnp.zeros((NP,), jnp.float32)
    zeros2 = jnp.zeros((NP, H2), jnp.float32)

    deg0, deg1 = _deg_call(ei3, ew2, zeros1)
    deg = deg0 + deg1
    deg16 = jnp.broadcast_to(deg.reshape(NP, 1), (NP, H2))
    degp = jnp.broadcast_to(deg.reshape(NPK, 8, 1),
                            (NPK, 8, H2)).reshape(NPK, 128)

    t1 = _enc_call(x, deg16, W1, b1.reshape(1, H1), W2, b2.reshape(1, H2),
                   Wg1)
    a10, a11 = _conv_call(t1, ei3, ew2, zeros2)
    t2p = _mid_call(a10.reshape(NPK, 128), a11.reshape(NPK, 128),
                    t1.reshape(NPK, 128), degp,
                    jnp.tile(bg1, 8).reshape(1, 128),
                    jnp.kron(jnp.eye(8, dtype=jnp.float32), Wg2))
    t2 = t2p.reshape(NP, H2)
    a20, a21 = _conv_call(t2, ei3, ew2, zeros2)
    mp2 = _mid2_call(a20.reshape(NPK, 128), a21.reshape(NPK, 128), t2p,
                     degp, jnp.tile(bg2, 8).reshape(1, 128))
    return _dec_call(mp2.reshape(NP, H2), W3,
                     b3.reshape(1, H1), W4, b4.reshape(1, D_IN))


# speedup vs baseline: 76.5634x; 1.0561x over previous
import functools

import jax
import jax.numpy as jnp
from jax import lax
from jax.experimental import pallas as pl
from jax.experimental.pallas import tpu as pltpu
from jax.experimental.pallas import tpu_sc as plsc

N = 100000
E = 3200000
D_IN = 128
H1 = 64
H2 = 16

NC = 2
NS = 16
NW = NC * NS

ER = E // 128
RB = ER // NW
RX = ER % NW

C = 512
K = C // 128
CG = C // 16
PAIRS = RB // (2 * K) * 1
PROWS = PAIRS * 2 * K

C2 = 1024
K2 = C2 // 128
PAIRS2 = RB // (2 * K2)
PROWS2 = PAIRS2 * 2 * K2

NP = 102400
NPK = NP // 8
RPS = NP // NS

_mesh = plsc.VectorSubcoreMesh(
    core_axis_name="c", subcore_axis_name="s", num_cores=NC, num_subcores=NS)


def _tile_rows(wid):
    extra = (wid < RX).astype(jnp.int32)
    row0 = wid * RB + jnp.minimum(wid, RX)
    return row0, RB + extra


@functools.partial(
    pl.kernel,
    out_type=(jax.ShapeDtypeStruct((NP,), jnp.float32),
              jax.ShapeDtypeStruct((NP,), jnp.float32)),
    mesh=_mesh,
    scratch_types=[
        pltpu.VMEM((K2, 128), jnp.int32),
        pltpu.VMEM((K2, 128), jnp.int32),
        pltpu.VMEM((K2, 128), jnp.float32),
        pltpu.VMEM((K2, 128), jnp.float32),
        pltpu.VMEM_SHARED((NP,), jnp.float32),
        pltpu.SemaphoreType.DMA,
        pltpu.SemaphoreType.DMA,
        pltpu.SemaphoreType.DMA,
        pltpu.SemaphoreType.DMA,
    ],
    compiler_params=pltpu.CompilerParams(use_tc_tiling_on_sc=False),
)
def _deg_call(ei_hbm, ew_hbm, zeros1_hbm, out0_hbm, out1_hbm,
              dst_a, dst_b, ew_a, ew_b, deg_sh, sia, sib, ssa, ssb):
    c = lax.axis_index("c")
    s = lax.axis_index("s")
    wid = s * NC + c
    row0, nrows = _tile_rows(wid)

    @pl.when(s == 0)
    def _():
        pltpu.sync_copy(zeros1_hbm, deg_sh)
    plsc.subcore_barrier()

    dst_src = ei_hbm.at[1]

    def pair(i2, carry):
        ra = row0 + i2 * (2 * K2)
        rb = ra + K2
        ia = [pltpu.async_copy(dst_src.at[pl.ds(ra, K2)], dst_a, sia),
              pltpu.async_copy(ew_hbm.at[pl.ds(ra, K2)], ew_a, sia)]
        ib = [pltpu.async_copy(dst_src.at[pl.ds(rb, K2)], dst_b, sib),
              pltpu.async_copy(ew_hbm.at[pl.ds(rb, K2)], ew_b, sib)]
        for d in ia:
            d.wait()
        sa = [pltpu.async_copy(ew_a.at[j], deg_sh.at[dst_a.at[j]], ssa,
                               add=True) for j in range(K2)]
        for d in ib:
            d.wait()
        sb = [pltpu.async_copy(ew_b.at[j], deg_sh.at[dst_b.at[j]], ssb,
                               add=True) for j in range(K2)]
        for d in sa:
            d.wait()
        for d in sb:
            d.wait()
        return carry

    lax.fori_loop(0, PAIRS2, pair, 0)

    def tail(r, carry):
        row = row0 + r
        pltpu.sync_copy(dst_src.at[row], dst_a.at[0])
        pltpu.sync_copy(ew_hbm.at[row], ew_a.at[0])
        pltpu.sync_copy(ew_a.at[0], deg_sh.at[dst_a.at[0]], add=True)
        return carry

    lax.fori_loop(PROWS2, nrows, tail, 0)
    plsc.subcore_barrier()

    @pl.when((s == 0) & (c == 0))
    def _():
        pltpu.sync_copy(deg_sh, out0_hbm)

    @pl.when((s == 0) & (c == 1))
    def _():
        pltpu.sync_copy(deg_sh, out1_hbm)


@functools.partial(
    pl.kernel,
    out_type=(jax.ShapeDtypeStruct((NP, H2), jnp.float32),
              jax.ShapeDtypeStruct((NP, H2), jnp.float32)),
    mesh=_mesh,
    scratch_types=[
        pltpu.VMEM((K, 128), jnp.int32),
        pltpu.VMEM((K, 128), jnp.int32),
        pltpu.VMEM((K, 128), jnp.int32),
        pltpu.VMEM((K, 128), jnp.int32),
        pltpu.VMEM((K, 128), jnp.float32),
        pltpu.VMEM((K, 128), jnp.float32),
        pltpu.VMEM((C, H2), jnp.float32),
        pltpu.VMEM((C, H2), jnp.float32),
        pltpu.VMEM_SHARED((NP, H2), jnp.float32),
        pltpu.SemaphoreType.DMA,
        pltpu.SemaphoreType.DMA,
        pltpu.SemaphoreType.DMA,
        pltpu.SemaphoreType.DMA,
        pltpu.SemaphoreType.DMA,
        pltpu.SemaphoreType.DMA,
    ],
    compiler_params=pltpu.CompilerParams(use_tc_tiling_on_sc=False),
)
def _conv_call(t_hbm, ei_hbm, ew_hbm, zeros2_hbm, out0_hbm, out1_hbm,
               src_a, src_b, dst_a, dst_b, ew_a, ew_b, rows_a, rows_b,
               agg_sh, sia, sib, sga, sgb, ssa, ssb):
    c = lax.axis_index("c")
    s = lax.axis_index("s")
    wid = s * NC + c
    row0, nrows = _tile_rows(wid)

    pltpu.sync_copy(zeros2_hbm.at[pl.ds(s * RPS, RPS)],
                    agg_sh.at[pl.ds(s * RPS, RPS)])
    plsc.subcore_barrier()

    src_src = ei_hbm.at[0]
    dst_src = ei_hbm.at[1]

    def _scale(rows_v, ew_v):
        @plsc.parallel_loop(0, K, 1, unroll=K)
        def _(j):
            for sub in range(8):
                ew16 = ew_v[j, pl.ds(sub * 16, 16)]
                base = j * 128 + sub * 16
                for l in range(16):
                    rows_v[base + l, :] = rows_v[base + l, :] * ew16[l]

    def pair(i2, carry):
        ra = row0 + i2 * (2 * K)
        rb = ra + K
        ia = [pltpu.async_copy(src_src.at[pl.ds(ra, K)], src_a, sia),
              pltpu.async_copy(dst_src.at[pl.ds(ra, K)], dst_a, sia),
              pltpu.async_copy(ew_hbm.at[pl.ds(ra, K)], ew_a, sia)]
        ib = [pltpu.async_copy(src_src.at[pl.ds(rb, K)], src_b, sib),
              pltpu.async_copy(dst_src.at[pl.ds(rb, K)], dst_b, sib),
              pltpu.async_copy(ew_hbm.at[pl.ds(rb, K)], ew_b, sib)]
        for d in ia:
            d.wait()
        ga = [pltpu.async_copy(t_hbm.at[src_a.at[j]],
                               rows_a.at[pl.ds(j * 128, 128)], sga)
              for j in range(K)]
        for d in ib:
            d.wait()
        gb = [pltpu.async_copy(t_hbm.at[src_b.at[j]],
                               rows_b.at[pl.ds(j * 128, 128)], sgb)
              for j in range(K)]
        for d in ga:
            d.wait()
        _scale(rows_a, ew_a)
        sa = [pltpu.async_copy(rows_a.at[pl.ds(j * 128, 128)],
                               agg_sh.at[dst_a.at[j]], ssa, add=True)
              for j in range(K)]
        for d in gb:
            d.wait()
        _scale(rows_b, ew_b)
        sb = [pltpu.async_copy(rows_b.at[pl.ds(j * 128, 128)],
                               agg_sh.at[dst_b.at[j]], ssb, add=True)
              for j in range(K)]
        for d in sa:
            d.wait()
        for d in sb:
            d.wait()
        return carry

    lax.fori_loop(0, PAIRS, pair, 0)

    def tail(r, carry):
        row = row0 + r
        pltpu.sync_copy(src_src.at[row], src_a.at[0])
        pltpu.sync_copy(dst_src.at[row], dst_a.at[0])
        pltpu.sync_copy(ew_hbm.at[row], ew_a.at[0])
        pltpu.async_copy(t_hbm.at[src_a.at[0]],
                         rows_a.at[pl.ds(0, 128)], sga).wait()
        for g in range(8):
            ew16 = ew_a[0, pl.ds(g * 16, 16)]
            for l in range(16):
                e = g * 16 + l
                rows_a[e, :] = rows_a[e, :] * ew16[l]
        pltpu.sync_copy(rows_a.at[pl.ds(0, 128)],
                        agg_sh.at[dst_a.at[0]], add=True)
        return carry

    lax.fori_loop(PROWS, nrows, tail, 0)
    plsc.subcore_barrier()

    @pl.when(c == 0)
    def _():
        pltpu.sync_copy(agg_sh.at[pl.ds(s * RPS, RPS)],
                        out0_hbm.at[pl.ds(s * RPS, RPS)])

    @pl.when(c == 1)
    def _():
        pltpu.sync_copy(agg_sh.at[pl.ds(s * RPS, RPS)],
                        out1_hbm.at[pl.ds(s * RPS, RPS)])


_R = 2000
_GRID = N // _R
_RP = 256


def _row_spec(cols):
    return pl.BlockSpec((_R, cols), lambda i: (i, 0))


def _pk_spec():
    return pl.BlockSpec((_RP, 128), lambda i: (i, 0))


def _full_spec(r, cols):
    return pl.BlockSpec((r, cols), lambda i: (0, 0))


def _enc_body(x_ref, d16_ref, w1_ref, b1_ref, w2_ref, b2_ref,
              wg1_ref, t_ref):
    h = jnp.maximum(
        jnp.dot(x_ref[...], w1_ref[...], preferred_element_type=jnp.float32)
        + b1_ref[...], 0.0)
    h = jnp.maximum(
        jnp.dot(h, w2_ref[...], preferred_element_type=jnp.float32)
        + b2_ref[...], 0.0)
    t = jnp.dot(h, wg1_ref[...], preferred_element_type=jnp.float32)
    t_ref[...] = t * lax.rsqrt(d16_ref[...] + 1.0)


def _enc_call(x, deg16, w1, b1, w2, b2, wg1):
    return pl.pallas_call(
        _enc_body,
        grid=(_GRID,),
        in_specs=[
            _row_spec(D_IN), _row_spec(H2),
            _full_spec(D_IN, H1), _full_spec(1, H1),
            _full_spec(H1, H2), _full_spec(1, H2),
            _full_spec(H2, H2),
        ],
        out_specs=_row_spec(H2),
        out_shape=jax.ShapeDtypeStruct((NP, H2), jnp.float32),
    )(x, deg16, w1, b1, w2, b2, wg1)


def _mid_body(a0_ref, a1_ref, t1_ref, dp_ref, bg1p_ref, wg2k_ref, t2_ref):
    dinv = lax.rsqrt(dp_ref[...] + 1.0)
    agg = a0_ref[...] + a1_ref[...] + t1_ref[...]
    m = jnp.maximum(agg * dinv + bg1p_ref[...], 0.0)
    u = jnp.dot(m, wg2k_ref[...], preferred_element_type=jnp.float32)
    t2_ref[...] = u * dinv


def _mid_call(a0p, a1p, t1p, degp, bg1p, wg2k):
    return pl.pallas_call(
        _mid_body,
        grid=(_GRID,),
        in_specs=[
            _pk_spec(), _pk_spec(), _pk_spec(), _pk_spec(),
            _full_spec(1, 128), _full_spec(128, 128),
        ],
        out_specs=_pk_spec(),
        out_shape=jax.ShapeDtypeStruct((NPK, 128), jnp.float32),
    )(a0p, a1p, t1p, degp, bg1p, wg2k)


def _mid2_body(a0_ref, a1_ref, t2_ref, dp_ref, bg2p_ref, m_ref):
    dinv = lax.rsqrt(dp_ref[...] + 1.0)
    agg = a0_ref[...] + a1_ref[...] + t2_ref[...]
    m_ref[...] = jnp.maximum(agg * dinv + bg2p_ref[...], 0.0)


def _mid2_call(a0p, a1p, t2p, degp, bg2p):
    return pl.pallas_call(
        _mid2_body,
        grid=(_GRID,),
        in_specs=[
            _pk_spec(), _pk_spec(), _pk_spec(), _pk_spec(),
            _full_spec(1, 128),
        ],
        out_specs=_pk_spec(),
        out_shape=jax.ShapeDtypeStruct((NPK, 128), jnp.float32),
    )(a0p, a1p, t2p, degp, bg2p)


def _dec_body(m_ref, w3_ref, b3_ref, w4_ref, b4_ref, y_ref):
    h = jnp.maximum(
        jnp.dot(m_ref[...], w3_ref[...],
                preferred_element_type=jnp.float32) + b3_ref[...], 0.0)
    y = jnp.dot(h, w4_ref[...], preferred_element_type=jnp.float32) + b4_ref[...]
    y_ref[...] = jax.nn.sigmoid(y)


def _dec_call(m, w3, b3, w4, b4):
    return pl.pallas_call(
        _dec_body,
        grid=(_GRID,),
        in_specs=[
            _row_spec(H2),
            _full_spec(H2, H1), _full_spec(1, H1),
            _full_spec(H1, D_IN), _full_spec(1, D_IN),
        ],
        out_specs=_row_spec(D_IN),
        out_shape=jax.ShapeDtypeStruct((N, D_IN), jnp.float32),
    )(m, w3, b3, w4, b4)


def kernel(x, edge_index, edge_weight, W1, b1, W2, b2, Wg1, bg1, Wg2, bg2,
           W3, b3, W4, b4):
    ei3 = edge_index.reshape(2, ER, 128)
    ew2 = edge_weight.reshape(ER, 128)
    zeros1 = jnp.zeros((NP,), jnp.float32)
    zeros2 = jnp.zeros((NP, H2), jnp.float32)

    deg0, deg1 = _deg_call(ei3, ew2, zeros1)
    deg = deg0 + deg1
    deg16 = jnp.broadcast_to(deg.reshape(NP, 1), (NP, H2))
    degp = jnp.broadcast_to(deg.reshape(NPK, 8, 1),
                            (NPK, 8, H2)).reshape(NPK, 128)

    t1 = _enc_call(x, deg16, W1, b1.reshape(1, H1), W2, b2.reshape(1, H2),
                   Wg1)
    a10, a11 = _conv_call(t1, ei3, ew2, zeros2)
    t2p = _mid_call(a10.reshape(NPK, 128), a11.reshape(NPK, 128),
                    t1.reshape(NPK, 128), degp,
                    jnp.tile(bg1, 8).reshape(1, 128),
                    jnp.kron(jnp.eye(8, dtype=jnp.float32), Wg2))
    t2 = t2p.reshape(NP, H2)
    a20, a21 = _conv_call(t2, ei3, ew2, zeros2)
    mp2 = _mid2_call(a20.reshape(NPK, 128), a21.reshape(NPK, 128), t2p,
                     degp, jnp.tile(bg2, 8).reshape(1, 128))
    return _dec_call(mp2.reshape(NP, H2), W3,
                     b3.reshape(1, H1), W4, b4.reshape(1, D_IN))
